# baseline (speedup 1.0000x reference)
import jax
import jax.numpy as jnp
from jax import lax
from jax.experimental import pallas as pl
from jax.experimental.pallas import tpu as pltpu
from jax.experimental.pallas import tpu_sc as plsc

VOCAB = 1000
BATCH = 1024
SEQ = 50
NC, NS = 2, 16
NW = NC * NS
NPIECE = 5
PCOLS = VOCAB // NPIECE
PVT = PCOLS // 8
NUNITS = SEQ * 8
MAXP = 13 * NPIECE
ROWB = VOCAB * 8
OUTW = (VOCAB // 8) * 8192


def _body(table5_hbm, idxt_hbm, out_hbm, idxcol, col5, gidx, rows, stage,
          gsem, ssem):
    w = lax.axis_index("s") * NC + lax.axis_index("c")
    iota = lax.iota(jnp.int32, 16)
    zeros = iota * 0

    @pl.loop(0, MAXP)
    def _(P):
        k = P // NPIECE
        p = P % NPIECE
        u = w + NW * k

        @pl.when(u < NUNITS)
        def _():
            t = u // 8
            bt = u % 8

            @pl.when(p == 0)
            def _():
                pltpu.sync_copy(
                    idxt_hbm.at[pl.ds(t, 1), pl.ds(128 * bt, 128)], idxcol
                )
                for g in range(8):
                    col5[pl.ds(16 * g, 16)] = idxcol[0, pl.ds(16 * g, 16)] * 5

            for g in range(8):
                gidx[pl.ds(16 * g, 16)] = col5[pl.ds(16 * g, 16)] + p

            pltpu.async_copy(table5_hbm.at[gidx], rows, gsem).wait()

            @pl.loop(0, 128)
            def _(bl):
                for g in range(12):
                    x = rows[bl, pl.ds(16 * g, 16)]
                    c = 16 * g + iota
                    pos = (c >> 3) * 1024 + (c & 7) * 128 + bl
                    plsc.store_scatter(stage, [pos], x)
                cc = jnp.minimum(192 + iota, PCOLS - 1)
                msk = iota < 8
                x = plsc.load_gather(rows, [zeros + bl, cc], mask=msk)
                pos = (cc >> 3) * 1024 + (cc & 7) * 128 + bl
                plsc.store_scatter(stage, [pos], x, mask=msk)

            base = (PVT * p) * 8192 + 1024 * bt
            for vtl in range(PVT):
                pltpu.async_copy(
                    stage.at[pl.ds(1024 * vtl, 1024)],
                    out_hbm.at[t, pl.ds(base + 8192 * vtl, 1024)],
                    ssem,
                )
            for vtl in range(PVT):
                pltpu.make_async_copy(
                    stage.at[pl.ds(1024 * vtl, 1024)],
                    out_hbm.at[t, pl.ds(base + 8192 * vtl, 1024)],
                    ssem,
                ).wait()


@jax.jit
def _lookup(idx, table):
    idxt = idx.T.astype(jnp.int32)
    table5 = table.reshape(VOCAB * NPIECE, PCOLS)
    mesh = plsc.VectorSubcoreMesh(core_axis_name="c", subcore_axis_name="s")
    run = pl.kernel(
        _body,
        out_type=jax.ShapeDtypeStruct((SEQ, OUTW), jnp.float32),
        mesh=mesh,
        compiler_params=pltpu.CompilerParams(
            use_tc_tiling_on_sc=False, needs_layout_passes=False
        ),
        scratch_types=[
            pltpu.VMEM((1, 128), jnp.int32),
            pltpu.VMEM((128,), jnp.int32),
            pltpu.VMEM((128,), jnp.int32),
            pltpu.VMEM((128, PCOLS), jnp.float32),
            pltpu.VMEM((PVT * 1024,), jnp.float32),
            pltpu.SemaphoreType.DMA,
            pltpu.SemaphoreType.DMA,
        ],
    )
    L = run(table5, idxt)
    L5 = L.reshape(SEQ, VOCAB // 8, 8, 8, 128)
    return L5.transpose(2, 4, 0, 1, 3).reshape(BATCH, SEQ, VOCAB)


def kernel(idx, table):
    return _lookup(idx, table)


# speedup vs baseline: 1.1834x; 1.1834x over previous
import jax
import jax.numpy as jnp
from jax import lax
from jax.experimental import pallas as pl
from jax.experimental.pallas import tpu as pltpu
from jax.experimental.pallas import tpu_sc as plsc

VOCAB = 1000
BATCH = 1024
SEQ = 50
NC, NS = 2, 16
NW = NC * NS
NPIECE = 5
PCOLS = VOCAB // NPIECE
PVT = PCOLS // 8
NUNITS = SEQ * 8
KMAX = 13
MAXP = KMAX * NPIECE
OUTW = (VOCAB // 8) * 8192


def _body(table5_hbm, idxu_hbm, out_hbm, units_v, col5, gidx, rows, stage,
          usem, gsem, ssem):
    w = lax.axis_index("s") * NC + lax.axis_index("c")
    iota = lax.iota(jnp.int32, 16)
    zeros = iota * 0
    npw = jnp.where(w < NUNITS - KMAX * NW + NW, MAXP, MAXP - NPIECE)

    uids = jnp.minimum(w + NW * iota, NUNITS - 1)
    units_v[pl.ds(0, 16)] = uids
    pltpu.async_copy(idxu_hbm.at[units_v], col5, usem).wait()
    for k in range(KMAX):
        for g in range(8):
            col5[k, pl.ds(16 * g, 16)] = col5[k, pl.ds(16 * g, 16)] * 5

    cvecs = [16 * g + iota for g in range(12)]
    pos0 = [(c >> 3) * 1024 + (c & 7) * 128 for c in cvecs]
    cc_t = jnp.minimum(192 + iota, PCOLS - 1)
    msk_t = iota < 8
    pos0_t = (cc_t >> 3) * 1024 + (cc_t & 7) * 128

    def prep_and_fire(P, b):
        k = P // NPIECE
        p = P % NPIECE
        for g in range(8):
            gidx[b][pl.ds(16 * g, 16)] = col5[k, pl.ds(16 * g, 16)] + p
        pltpu.async_copy(table5_hbm.at[gidx[b]], rows[b], gsem[b])

    def wait_gather(b):
        pltpu.make_async_copy(
            table5_hbm.at[gidx[b]], rows[b], gsem[b]
        ).wait()

    def fire_stores(P, b):
        k = P // NPIECE
        p = P % NPIECE
        u = w + NW * k
        t = u // 8
        bt = u % 8
        base = (PVT * p) * 8192 + 1024 * bt
        for vtl in range(PVT):
            pltpu.async_copy(
                stage[b].at[pl.ds(1024 * vtl, 1024)],
                out_hbm.at[t, pl.ds(base + 8192 * vtl, 1024)],
                ssem[b],
            )

    def drain_stores(b):
        for vtl in range(PVT):
            pltpu.make_async_copy(
                stage[b].at[pl.ds(1024 * vtl, 1024)],
                out_hbm.at[0, pl.ds(8192 * vtl, 1024)],
                ssem[b],
            ).wait()

    def transpose(b):
        @pl.loop(0, 128, step=2)
        def _(bl0):
            for d in range(2):
                bl = bl0 + d
                for g in range(12):
                    x = rows[b][bl, pl.ds(16 * g, 16)]
                    plsc.store_scatter(stage[b], [pos0[g] + bl], x)
                x = plsc.load_gather(rows[b], [zeros + bl, cc_t], mask=msk_t)
                plsc.store_scatter(stage[b], [pos0_t + bl], x, mask=msk_t)

    prep_and_fire(0, 0)

    @pl.loop(0, MAXP + 1, step=2)
    def _(P0):
        for d in range(2):
            P = P0 + d
            b = d

            @pl.when(P < npw)
            def _():
                wait_gather(b)

                @pl.when(P + 1 < npw)
                def _():
                    prep_and_fire(P + 1, 1 - b)

                @pl.when(P >= 2)
                def _():
                    drain_stores(b)

                transpose(b)
                fire_stores(P, b)

    drain_stores(0)
    drain_stores(1)


@jax.jit
def _lookup(idx, table):
    idxu = idx.T.reshape(SEQ * 8, 128).astype(jnp.int32)
    table5 = table.reshape(VOCAB * NPIECE, PCOLS)
    mesh = plsc.VectorSubcoreMesh(core_axis_name="c", subcore_axis_name="s")
    run = pl.kernel(
        _body,
        out_type=jax.ShapeDtypeStruct((SEQ, OUTW), jnp.float32),
        mesh=mesh,
        compiler_params=pltpu.CompilerParams(
            use_tc_tiling_on_sc=False, needs_layout_passes=False
        ),
        scratch_types=[
            pltpu.VMEM((16,), jnp.int32),
            pltpu.VMEM((16, 128), jnp.int32),
            [pltpu.VMEM((128,), jnp.int32) for _ in range(2)],
            [pltpu.VMEM((128, PCOLS), jnp.float32) for _ in range(2)],
            [pltpu.VMEM((PVT * 1024,), jnp.float32) for _ in range(2)],
            pltpu.SemaphoreType.DMA,
            [pltpu.SemaphoreType.DMA for _ in range(2)],
            [pltpu.SemaphoreType.DMA for _ in range(2)],
        ],
    )
    L = run(table5, idxu)
    L5 = L.reshape(SEQ, VOCAB // 8, 8, 8, 128)
    return L5.transpose(2, 4, 0, 1, 3).reshape(BATCH, SEQ, VOCAB)


def kernel(idx, table):
    return _lookup(idx, table)


# speedup vs baseline: 2.6417x; 2.2323x over previous
import jax
import jax.numpy as jnp
from jax import lax
from jax.experimental import pallas as pl
from jax.experimental.pallas import tpu as pltpu
from jax.experimental.pallas import tpu_sc as plsc

VOCAB = 1000
BATCH = 1024
SEQ = 50
NC, NS = 2, 16
NW = NC * NS
NPIECE = 5
PCOLS = VOCAB // NPIECE
PVT = PCOLS // 8
NUNITS = SEQ * 8
KMAX = 13
MAXP = KMAX * NPIECE
OUTW = (VOCAB // 8) * 8192


def _body(table5_hbm, idxu_hbm, out_hbm, units_v, col5, gidx, rows, stage,
          usem, gsem, ssem):
    w = lax.axis_index("s") * NC + lax.axis_index("c")
    iota = lax.iota(jnp.int32, 16)
    zeros = iota * 0
    npw = jnp.where(w < NUNITS - KMAX * NW + NW, MAXP, MAXP - NPIECE)

    uids = jnp.minimum(w + NW * iota, NUNITS - 1)
    units_v[pl.ds(0, 16)] = uids
    pltpu.async_copy(idxu_hbm.at[units_v], col5, usem).wait()
    for k in range(KMAX):
        for g in range(8):
            col5[k, pl.ds(16 * g, 16)] = col5[k, pl.ds(16 * g, 16)] * 5

    diag = [(iota + r) & 15 for r in range(16)]
    qdiag = [(d >> 3) * 1024 + (d & 7) * 128 + iota for d in diag]
    diag_t = [jnp.minimum(192 + d, PCOLS - 1) for d in diag]
    msk_t = [d < 8 for d in diag]

    def prep_and_fire(P, b):
        k = P // NPIECE
        p = P % NPIECE
        for g in range(8):
            gidx[b][pl.ds(16 * g, 16)] = col5[k, pl.ds(16 * g, 16)] + p
        pltpu.async_copy(table5_hbm.at[gidx[b]], rows[b], gsem[b])

    def wait_gather(b):
        pltpu.make_async_copy(
            table5_hbm.at[gidx[b]], rows[b], gsem[b]
        ).wait()

    def fire_stores(P, b):
        k = P // NPIECE
        p = P % NPIECE
        u = w + NW * k
        t = u // 8
        bt = u % 8
        base = (PVT * p) * 8192 + 1024 * bt
        for vtl in range(PVT):
            pltpu.async_copy(
                stage[b].at[pl.ds(1024 * vtl, 1024)],
                out_hbm.at[t, pl.ds(base + 8192 * vtl, 1024)],
                ssem[b],
            )

    def drain_stores(b):
        for vtl in range(PVT):
            pltpu.make_async_copy(
                stage[b].at[pl.ds(1024 * vtl, 1024)],
                out_hbm.at[0, pl.ds(8192 * vtl, 1024)],
                ssem[b],
            ).wait()

    def transpose(b):
        @pl.loop(0, 128, step=16)
        def _(bl0):
            rowv = iota + bl0

            @pl.loop(0, 192, step=16)
            def _(c0):
                s = c0 * 128 + bl0
                for r in range(16):
                    x = plsc.load_gather(rows[b], [rowv, diag[r] + c0])
                    plsc.store_scatter(stage[b], [qdiag[r] + s], x)

            s = 192 * 128 + bl0
            for r in range(16):
                x = plsc.load_gather(rows[b], [rowv, diag_t[r]],
                                     mask=msk_t[r])
                plsc.store_scatter(stage[b], [qdiag[r] + s], x,
                                   mask=msk_t[r])

    prep_and_fire(0, 0)

    @pl.loop(0, MAXP + 1, step=2)
    def _(P0):
        for d in range(2):
            P = P0 + d
            b = d

            @pl.when(P < npw)
            def _():
                wait_gather(b)

                @pl.when(P + 1 < npw)
                def _():
                    prep_and_fire(P + 1, 1 - b)

                @pl.when(P >= 2)
                def _():
                    drain_stores(b)

                transpose(b)
                fire_stores(P, b)

    drain_stores(0)
    drain_stores(1)


@jax.jit
def _lookup(idx, table):
    idxu = idx.T.reshape(SEQ * 8, 128).astype(jnp.int32)
    table5 = table.reshape(VOCAB * NPIECE, PCOLS)
    mesh = plsc.VectorSubcoreMesh(core_axis_name="c", subcore_axis_name="s")
    run = pl.kernel(
        _body,
        out_type=jax.ShapeDtypeStruct((SEQ, OUTW), jnp.float32),
        mesh=mesh,
        compiler_params=pltpu.CompilerParams(
            use_tc_tiling_on_sc=False, needs_layout_passes=False
        ),
        scratch_types=[
            pltpu.VMEM((16,), jnp.int32),
            pltpu.VMEM((16, 128), jnp.int32),
            [pltpu.VMEM((128,), jnp.int32) for _ in range(2)],
            [pltpu.VMEM((128, PCOLS), jnp.float32) for _ in range(2)],
            [pltpu.VMEM((PVT * 1024,), jnp.float32) for _ in range(2)],
            pltpu.SemaphoreType.DMA,
            [pltpu.SemaphoreType.DMA for _ in range(2)],
            [pltpu.SemaphoreType.DMA for _ in range(2)],
        ],
    )
    L = run(table5, idxu)
    L5 = L.reshape(SEQ, VOCAB // 8, 8, 8, 128)
    return L5.transpose(2, 4, 0, 1, 3).reshape(BATCH, SEQ, VOCAB)


def kernel(idx, table):
    return _lookup(idx, table)


# speedup vs baseline: 6.1890x; 2.3428x over previous
import jax
import jax.numpy as jnp
from jax import lax
from jax.experimental import pallas as pl
from jax.experimental.pallas import tpu as pltpu
from jax.experimental.pallas import tpu_sc as plsc

VOCAB = 1000
BATCH = 1024
SEQ = 50
NC, NS = 2, 16
NW = NC * NS
NPIECE = 5
PCOLS = VOCAB // NPIECE
PVT = PCOLS // 8
NUNITS = SEQ * 8
KMAX = 13
MAXP = KMAX * NPIECE
OUTW = (VOCAB // 8) * 8192


def _body(table5_hbm, idxu_hbm, out_hbm, units_v, col5, gidx, rows, stage,
          usem, gsem, ssem):
    w = lax.axis_index("s") * NC + lax.axis_index("c")
    iota = lax.iota(jnp.int32, 16)
    zeros = iota * 0
    npw = jnp.where(w < NUNITS - KMAX * NW + NW, MAXP, MAXP - NPIECE)

    uids = jnp.minimum(w + NW * iota, NUNITS - 1)
    units_v[pl.ds(0, 16)] = uids
    pltpu.async_copy(idxu_hbm.at[units_v], col5, usem).wait()
    for k in range(KMAX):
        for g in range(8):
            col5[k, pl.ds(16 * g, 16)] = col5[k, pl.ds(16 * g, 16)] * 5

    diag = [(iota + r) & 15 for r in range(16)]
    qdiag = [(d >> 3) * 1024 + (d & 7) * 128 + iota for d in diag]
    diag_t = [jnp.minimum(192 + d, PCOLS - 1) for d in diag]
    msk_t = [d < 8 for d in diag]

    def prep_and_fire(P, b):
        k = P // NPIECE
        p = P % NPIECE
        for g in range(8):
            gidx[b][pl.ds(16 * g, 16)] = col5[k, pl.ds(16 * g, 16)] + p
        pltpu.async_copy(table5_hbm.at[gidx[b]], rows[b], gsem[b])

    def wait_gather(b):
        pltpu.make_async_copy(
            table5_hbm.at[gidx[b]], rows[b], gsem[b]
        ).wait()

    def fire_stores(P, b):
        k = P // NPIECE
        p = P % NPIECE
        u = w + NW * k
        t = u // 8
        bt = u % 8
        base = (PVT * p) * 8192 + 1024 * bt
        for vtl in range(PVT):
            pltpu.async_copy(
                stage[b].at[pl.ds(1024 * vtl, 1024)],
                out_hbm.at[t, pl.ds(base + 8192 * vtl, 1024)],
                ssem[b],
            )

    def drain_stores(b):
        for vtl in range(PVT):
            pltpu.make_async_copy(
                stage[b].at[pl.ds(1024 * vtl, 1024)],
                out_hbm.at[0, pl.ds(8192 * vtl, 1024)],
                ssem[b],
            ).wait()

    def transpose(b):
        @plsc.parallel_loop(0, 128, step=16)
        def _(bl0):
            rowv = iota + bl0

            @plsc.parallel_loop(0, 192, step=16, unroll=2)
            def _(c0):
                s = c0 * 128 + bl0
                for r in range(16):
                    x = plsc.load_gather(rows[b], [rowv, diag[r] + c0])
                    plsc.store_scatter(stage[b], [qdiag[r] + s], x)

            s = 192 * 128 + bl0
            for r in range(16):
                x = plsc.load_gather(rows[b], [rowv, diag_t[r]],
                                     mask=msk_t[r])
                plsc.store_scatter(stage[b], [qdiag[r] + s], x,
                                   mask=msk_t[r])

    prep_and_fire(0, 0)

    @pl.loop(0, MAXP + 1, step=2)
    def _(P0):
        for d in range(2):
            P = P0 + d
            b = d

            @pl.when(P < npw)
            def _():
                wait_gather(b)

                @pl.when(P + 1 < npw)
                def _():
                    prep_and_fire(P + 1, 1 - b)

                @pl.when(P >= 2)
                def _():
                    drain_stores(b)

                transpose(b)
                fire_stores(P, b)

    drain_stores(0)
    drain_stores(1)


@jax.jit
def _lookup(idx, table):
    idxu = idx.T.reshape(SEQ * 8, 128).astype(jnp.int32)
    table5 = table.reshape(VOCAB * NPIECE, PCOLS)
    mesh = plsc.VectorSubcoreMesh(core_axis_name="c", subcore_axis_name="s")
    run = pl.kernel(
        _body,
        out_type=jax.ShapeDtypeStruct((SEQ, OUTW), jnp.float32),
        mesh=mesh,
        compiler_params=pltpu.CompilerParams(
            use_tc_tiling_on_sc=False, needs_layout_passes=False
        ),
        scratch_types=[
            pltpu.VMEM((16,), jnp.int32),
            pltpu.VMEM((16, 128), jnp.int32),
            [pltpu.VMEM((128,), jnp.int32) for _ in range(2)],
            [pltpu.VMEM((128, PCOLS), jnp.float32) for _ in range(2)],
            [pltpu.VMEM((PVT * 1024,), jnp.float32) for _ in range(2)],
            pltpu.SemaphoreType.DMA,
            [pltpu.SemaphoreType.DMA for _ in range(2)],
            [pltpu.SemaphoreType.DMA for _ in range(2)],
        ],
    )
    L = run(table5, idxu)
    L5 = L.reshape(SEQ, VOCAB // 8, 8, 8, 128)
    return L5.transpose(2, 4, 0, 1, 3).reshape(BATCH, SEQ, VOCAB)


def kernel(idx, table):
    return _lookup(idx, table)
